# Initial kernel scaffold; baseline (speedup 1.0000x reference)
import jax
import jax.numpy as jnp
from jax.experimental import pallas as pl


def kernel(x, Wq, Wk, Wv):
    raise NotImplementedError("write your pallas kernel here")


# speedup vs baseline: 6.8790x; 6.8790x over previous
import functools

import jax
import jax.numpy as jnp
from jax.experimental import pallas as pl

_T = 64
_C = 128
_K = 3


def _body(x_ref, wq_ref, wk_ref, wv_ref, o_ref, *, g):
    xb = x_ref[...]
    x2 = xb.reshape(g * _T, _C)
    wq = wq_ref[...]
    wk = wk_ref[...]
    wv = wv_ref[...]
    q = jnp.dot(x2, wq, preferred_element_type=jnp.float32).reshape(g, _T, _C)
    k = jnp.dot(x2, wk, preferred_element_type=jnp.float32).reshape(g, _T, _C)
    v = jnp.dot(x2, wv, preferred_element_type=jnp.float32).reshape(g, _T, _C)

    e = jax.lax.dot_general(
        q, k, (((2,), (2,)), ((0,), (0,))),
        preferred_element_type=jnp.float32)

    iota = jax.lax.broadcasted_iota(jnp.int32, e.shape, 2)
    neg_inf = jnp.float32(float("-inf"))

    def pick(e_cur):
        m = jnp.max(e_cur, axis=-1, keepdims=True)
        j = jnp.min(jnp.where(e_cur == m, iota, _T), axis=-1, keepdims=True)
        sel = iota == j
        return sel, jnp.where(sel, neg_inf, e_cur)

    s1, e1 = pick(e)
    s2, e2 = pick(e1)
    s3, _ = pick(e2)
    sel = s1 | s2 | s3
    score = jnp.where(sel, jax.nn.relu(e), jnp.float32(0.0))

    out = jax.lax.dot_general(
        score, v, (((2,), (1,)), ((0,), (0,))),
        preferred_element_type=jnp.float32)
    o_ref[...] = out


@jax.jit
def kernel(x, Wq, Wk, Wv):
    B, N, T, C = x.shape
    S = B * N
    g = 16
    xs = x.reshape(S, T, C)
    out = pl.pallas_call(
        functools.partial(_body, g=g),
        grid=(S // g,),
        in_specs=[
            pl.BlockSpec((g, T, C), lambda i: (i, 0, 0)),
            pl.BlockSpec((C, C), lambda i: (0, 0)),
            pl.BlockSpec((C, C), lambda i: (0, 0)),
            pl.BlockSpec((C, C), lambda i: (0, 0)),
        ],
        out_specs=pl.BlockSpec((g, T, C), lambda i: (i, 0, 0)),
        out_shape=jax.ShapeDtypeStruct((S, T, C), jnp.float32),
    )(xs, Wq, Wk, Wv)
    return out.reshape(B, N, T, C)


# speedup vs baseline: 11.5541x; 1.6796x over previous
import functools

import jax
import jax.numpy as jnp
from jax.experimental import pallas as pl

_T = 64
_C = 128
_K = 3


def _body(x_ref, wq_ref, wk_ref, wv_ref, o_ref, *, g):
    xb = x_ref[...]
    x2 = xb.reshape(g * _T, _C)
    wq = wq_ref[...]
    wk = wk_ref[...]
    wv = wv_ref[...]
    q = jnp.dot(x2, wq, preferred_element_type=jnp.float32).reshape(g, _T, _C)
    k = jnp.dot(x2, wk, preferred_element_type=jnp.float32).reshape(g, _T, _C)
    v = jnp.dot(x2, wv, preferred_element_type=jnp.float32).reshape(g, _T, _C)

    e = jax.lax.dot_general(
        q, k, (((2,), (2,)), ((0,), (0,))),
        preferred_element_type=jnp.float32)

    neg_inf = jnp.float32(float("-inf"))
    m1 = jnp.max(e, axis=-1, keepdims=True)
    e1 = jnp.where(e == m1, neg_inf, e)
    m2 = jnp.max(e1, axis=-1, keepdims=True)
    e2 = jnp.where(e1 == m2, neg_inf, e1)
    m3 = jnp.max(e2, axis=-1, keepdims=True)
    score = jnp.where(e >= m3, jax.nn.relu(e), jnp.float32(0.0))

    out = jax.lax.dot_general(
        score, v, (((2,), (1,)), ((0,), (0,))),
        preferred_element_type=jnp.float32)
    o_ref[...] = out


@jax.jit
def kernel(x, Wq, Wk, Wv):
    B, N, T, C = x.shape
    S = B * N
    g = 16
    xs = x.reshape(S, T, C)
    out = pl.pallas_call(
        functools.partial(_body, g=g),
        grid=(S // g,),
        in_specs=[
            pl.BlockSpec((g, T, C), lambda i: (i, 0, 0)),
            pl.BlockSpec((C, C), lambda i: (0, 0)),
            pl.BlockSpec((C, C), lambda i: (0, 0)),
            pl.BlockSpec((C, C), lambda i: (0, 0)),
        ],
        out_specs=pl.BlockSpec((g, T, C), lambda i: (i, 0, 0)),
        out_shape=jax.ShapeDtypeStruct((S, T, C), jnp.float32),
    )(xs, Wq, Wk, Wv)
    return out.reshape(B, N, T, C)


# speedup vs baseline: 12.1293x; 1.0498x over previous
import functools

import jax
import jax.numpy as jnp
from jax.experimental import pallas as pl

_T = 64
_C = 128


def _mm_body(a_ref, b_ref, o_ref):
    o_ref[...] = jax.lax.dot_general(
        a_ref[...], b_ref[...], (((1,), (1,)), ((), ())),
        preferred_element_type=jnp.float32)


def _body(x_ref, m_ref, wv_ref, o_ref, *, g):
    xb = x_ref[...]
    x2 = xb.reshape(g * _T, _C)
    xm = jnp.dot(x2, m_ref[...], preferred_element_type=jnp.float32)
    xm = xm.reshape(g, _T, _C)
    v = jnp.dot(x2, wv_ref[...], preferred_element_type=jnp.float32)
    v = v.reshape(g, _T, _C)

    et = jax.lax.dot_general(
        xb, xm, (((2,), (2,)), ((0,), (0,))),
        preferred_element_type=jnp.float32)

    neg_inf = jnp.float32(float("-inf"))
    m1 = jnp.max(et, axis=1, keepdims=True)
    e1 = jnp.where(et == m1, neg_inf, et)
    m2 = jnp.max(e1, axis=1, keepdims=True)
    e2 = jnp.where(e1 == m2, neg_inf, e1)
    m3 = jnp.max(e2, axis=1, keepdims=True)
    score = jnp.where(et >= m3, jax.nn.relu(et), jnp.float32(0.0))

    out = jax.lax.dot_general(
        score, v, (((1,), (1,)), ((0,), (0,))),
        preferred_element_type=jnp.float32)
    o_ref[...] = out


@jax.jit
def kernel(x, Wq, Wk, Wv):
    B, N, T, C = x.shape
    S = B * N
    g = 16
    M = pl.pallas_call(
        _mm_body,
        out_shape=jax.ShapeDtypeStruct((C, C), jnp.float32),
    )(Wq, Wk)
    xs = x.reshape(S, T, C)
    out = pl.pallas_call(
        functools.partial(_body, g=g),
        grid=(S // g,),
        in_specs=[
            pl.BlockSpec((g, T, C), lambda i: (i, 0, 0)),
            pl.BlockSpec((C, C), lambda i: (0, 0)),
            pl.BlockSpec((C, C), lambda i: (0, 0)),
        ],
        out_specs=pl.BlockSpec((g, T, C), lambda i: (i, 0, 0)),
        out_shape=jax.ShapeDtypeStruct((S, T, C), jnp.float32),
    )(xs, M, Wv)
    return out.reshape(B, N, T, C)


# speedup vs baseline: 12.3618x; 1.0192x over previous
import functools

import jax
import jax.numpy as jnp
from jax.experimental import pallas as pl

_T = 64
_C = 128
_K = 3


def _body(x_ref, wq_ref, wk_ref, wv_ref, o_ref, *, g):
    xb = x_ref[...]
    x2 = xb.reshape(g * _T, _C)
    wq = wq_ref[...]
    wk = wk_ref[...]
    wv = wv_ref[...]
    q = jnp.dot(x2, wq, preferred_element_type=jnp.float32).reshape(g, _T, _C)
    k = jnp.dot(x2, wk, preferred_element_type=jnp.float32).reshape(g, _T, _C)
    v = jnp.dot(x2, wv, preferred_element_type=jnp.float32).reshape(g, _T, _C)

    et = jax.lax.dot_general(
        k, q, (((2,), (2,)), ((0,), (0,))),
        preferred_element_type=jnp.float32)

    neg_inf = jnp.float32(float("-inf"))
    m1 = jnp.max(et, axis=1, keepdims=True)
    e1 = jnp.where(et == m1, neg_inf, et)
    m2 = jnp.max(e1, axis=1, keepdims=True)
    e2 = jnp.where(e1 == m2, neg_inf, e1)
    m3 = jnp.max(e2, axis=1, keepdims=True)
    score = jnp.where(et >= m3, jax.nn.relu(et), jnp.float32(0.0))

    out = jax.lax.dot_general(
        score, v, (((1,), (1,)), ((0,), (0,))),
        preferred_element_type=jnp.float32)
    o_ref[...] = out


@jax.jit
def kernel(x, Wq, Wk, Wv):
    B, N, T, C = x.shape
    S = B * N
    g = 16
    xs = x.reshape(S, T, C)
    out = pl.pallas_call(
        functools.partial(_body, g=g),
        grid=(S // g,),
        in_specs=[
            pl.BlockSpec((g, T, C), lambda i: (i, 0, 0)),
            pl.BlockSpec((C, C), lambda i: (0, 0)),
            pl.BlockSpec((C, C), lambda i: (0, 0)),
            pl.BlockSpec((C, C), lambda i: (0, 0)),
        ],
        out_specs=pl.BlockSpec((g, T, C), lambda i: (i, 0, 0)),
        out_shape=jax.ShapeDtypeStruct((S, T, C), jnp.float32),
    )(xs, Wq, Wk, Wv)
    return out.reshape(B, N, T, C)


# speedup vs baseline: 25.9691x; 2.1008x over previous
import functools

import jax
import jax.numpy as jnp
from jax.experimental import pallas as pl

_T = 64
_C = 128
_K = 3


def _body(x_ref, wq_ref, wk_ref, wv_ref, o_ref, *, g):
    xb = x_ref[...]
    x2 = xb.reshape(g * _T, _C)
    wq = wq_ref[...]
    wk = wk_ref[...]
    wv = wv_ref[...]
    q = jnp.dot(x2, wq, preferred_element_type=jnp.float32).reshape(g, _T, _C)
    k = jnp.dot(x2, wk, preferred_element_type=jnp.float32).reshape(g, _T, _C)
    v = jnp.dot(x2, wv, preferred_element_type=jnp.float32).reshape(g, _T, _C)

    et = jax.lax.dot_general(
        k, q, (((2,), (2,)), ((0,), (0,))),
        preferred_element_type=jnp.float32)

    neg_inf = jnp.float32(float("-inf"))
    m1 = jnp.max(et, axis=1, keepdims=True)
    e1 = jnp.where(et == m1, neg_inf, et)
    m2 = jnp.max(e1, axis=1, keepdims=True)
    e2 = jnp.where(e1 == m2, neg_inf, e1)
    m3 = jnp.max(e2, axis=1, keepdims=True)
    score = jnp.where(et >= m3, jax.nn.relu(et), jnp.float32(0.0))

    out = jax.lax.dot_general(
        score, v, (((1,), (1,)), ((0,), (0,))),
        preferred_element_type=jnp.float32)
    o_ref[...] = out


@jax.jit
def kernel(x, Wq, Wk, Wv):
    B, N, T, C = x.shape
    S = B * N
    g = 104
    xs = x.reshape(S, T, C)
    out = pl.pallas_call(
        functools.partial(_body, g=g),
        grid=(S // g,),
        in_specs=[
            pl.BlockSpec((g, T, C), lambda i: (i, 0, 0)),
            pl.BlockSpec((C, C), lambda i: (0, 0)),
            pl.BlockSpec((C, C), lambda i: (0, 0)),
            pl.BlockSpec((C, C), lambda i: (0, 0)),
        ],
        out_specs=pl.BlockSpec((g, T, C), lambda i: (i, 0, 0)),
        out_shape=jax.ShapeDtypeStruct((S, T, C), jnp.float32),
    )(xs, Wq, Wk, Wv)
    return out.reshape(B, N, T, C)


# speedup vs baseline: 26.6468x; 1.0261x over previous
import functools

import jax
import jax.numpy as jnp
from jax.experimental import pallas as pl

_T = 64
_C = 128
_K = 3


def _body(x_ref, wq_ref, wk_ref, wv_ref, o_ref, *, g):
    xb = x_ref[...]
    x2 = xb.reshape(g * _T, _C)
    wq = wq_ref[...]
    wk = wk_ref[...]
    wv = wv_ref[...]
    q = jnp.dot(x2, wq, preferred_element_type=jnp.float32).reshape(g, _T, _C)
    k = jnp.dot(x2, wk, preferred_element_type=jnp.float32).reshape(g, _T, _C)
    v = jnp.dot(x2, wv, preferred_element_type=jnp.float32).reshape(g, _T, _C)

    et = jax.lax.dot_general(
        k, q, (((2,), (2,)), ((0,), (0,))),
        preferred_element_type=jnp.float32)

    neg_inf = jnp.float32(float("-inf"))
    m1 = jnp.max(et, axis=1, keepdims=True)
    e1 = jnp.where(et == m1, neg_inf, et)
    m2 = jnp.max(e1, axis=1, keepdims=True)
    e2 = jnp.where(e1 == m2, neg_inf, e1)
    m3 = jnp.max(e2, axis=1, keepdims=True)
    score = jnp.where(et >= m3, jax.nn.relu(et), jnp.float32(0.0))

    out = jax.lax.dot_general(
        score, v, (((1,), (1,)), ((0,), (0,))),
        preferred_element_type=jnp.float32)
    o_ref[...] = out


@jax.jit
def kernel(x, Wq, Wk, Wv):
    B, N, T, C = x.shape
    S = B * N
    g = 200
    xs = x.reshape(S, T, C)
    out = pl.pallas_call(
        functools.partial(_body, g=g),
        grid=(S // g,),
        in_specs=[
            pl.BlockSpec((g, T, C), lambda i: (i, 0, 0)),
            pl.BlockSpec((C, C), lambda i: (0, 0)),
            pl.BlockSpec((C, C), lambda i: (0, 0)),
            pl.BlockSpec((C, C), lambda i: (0, 0)),
        ],
        out_specs=pl.BlockSpec((g, T, C), lambda i: (i, 0, 0)),
        out_shape=jax.ShapeDtypeStruct((S, T, C), jnp.float32),
    )(xs, Wq, Wk, Wv)
    return out.reshape(B, N, T, C)
